# baseline (speedup 1.0000x reference)
import functools

import jax
import jax.numpy as jnp
from jax import lax
from jax.experimental import pallas as pl
from jax.experimental.pallas import tpu as pltpu
from jax.experimental.pallas import tpu_sc as plsc

N_SUB = 8
N_NODES = 1250
N_CONV = N_NODES - 1
E_SUB = 20000
D = 256
NP = 1280
XR = 1256
CFLAT = NP * NP
NCORES = 2
NTILES = 16
CHUNK = 1248
EBUF = 1280
EWIN = EBUF + 128
NIDX = 128
DIAG_PER_TILE = NP // NTILES
STRIPE = CFLAT // NTILES
STRIPE_ROWS = NP // NTILES
ZCHUNK = 12800
SG_SPLIT = 4
RB = 256
NRB = NP // RB
SG_PER_CORE = SG_SPLIT // NCORES
DEGROW = NP - 1


def _sc_counts_body(sg_base, edges_hbm, out_hbm, ev_v, flat_v, val_v,
                    gflat_v, dflat_v, gdflat_v, dval_v, zero_v, c_sh, out_sem,
                    zsem, esem, ssem):
    cid = lax.axis_index("c")
    sid = lax.axis_index("s")
    iota16 = lax.iota(jnp.int32, 16)

    nvalid = jnp.where(sid == NTILES - 1, E_SUB - (NTILES - 1) * CHUNK, CHUNK)

    def _fill_vals(k, _):
        lanes = k * 16 + iota16
        val_v[k // 8, pl.ds((k % 8) * 16, 16)] = jnp.where(
            lanes < nvalid, 1.0, 0.0).astype(jnp.float32)
        return 0
    lax.fori_loop(0, EBUF // 16, _fill_vals, 0)

    def _fill_dvals(k, _):
        lanes = k * 16 + iota16
        dnode = sid * DIAG_PER_TILE + lanes
        dval_v[0, pl.ds(k * 16, 16)] = jnp.where(
            (lanes < DIAG_PER_TILE) & (dnode < N_CONV), 1.0,
            0.0).astype(jnp.float32)
        return 0
    lax.fori_loop(0, NIDX // 16, _fill_dvals, 0)

    def _fill_zero(k, _):
        zero_v[pl.ds(k * 16, 16)] = jnp.zeros((16,), jnp.float32)
        return 0
    lax.fori_loop(0, ZCHUNK // 16, _fill_zero, 0)

    stripe_lo = sid * STRIPE

    def _one_subgraph(g, _):
        i = sg_base + cid + NCORES * g
        node_lo = i * N_NODES
        rshift = lax.rem(node_lo, 8)
        ebase = i * E_SUB + sid * CHUNK
        ebase_al = jnp.minimum(ebase // 128 * 128, N_SUB * E_SUB - EWIN)
        eoff = ebase - ebase_al

        def _zero(j, _):
            pltpu.async_copy(
                zero_v, c_sh.at[pl.ds(stripe_lo + j * ZCHUNK, ZCHUNK)], zsem)
            return 0
        lax.fori_loop(0, STRIPE // ZCHUNK, _zero, 0)
        pltpu.async_copy(edges_hbm.at[:, pl.ds(ebase_al, EWIN)], ev_v, esem)
        pltpu.make_async_copy(edges_hbm.at[:, pl.ds(ebase_al, EWIN)], ev_v,
                              esem).wait()

        def _flat(k, _):
            sl = ev_v[0, pl.ds(eoff + k * 16, 16)] - node_lo
            dl = ev_v[1, pl.ds(eoff + k * 16, 16)] - node_lo
            flat_v[k // 8, pl.ds((k % 8) * 16, 16)] = dl * NP + sl + rshift
            gflat_v[k // 8, pl.ds((k % 8) * 16, 16)] = \
                DEGROW * NP + dl + rshift
            return 0
        lax.fori_loop(0, EBUF // 16, _flat, 0)

        def _fill_dflat(k, _):
            dnode = sid * DIAG_PER_TILE + k * 16 + iota16
            dflat_v[0, pl.ds(k * 16, 16)] = jnp.where(
                dnode < N_CONV, dnode * (NP + 1) + rshift, 0)
            gdflat_v[0, pl.ds(k * 16, 16)] = jnp.where(
                dnode < N_CONV, DEGROW * NP + dnode + rshift, 0)
            return 0
        lax.fori_loop(0, NIDX // 16, _fill_dflat, 0)

        def _zdrain(j, _):
            pltpu.make_async_copy(
                zero_v, c_sh.at[pl.ds(stripe_lo + j * ZCHUNK, ZCHUNK)],
                zsem).wait()
            return 0
        lax.fori_loop(0, STRIPE // ZCHUNK, _zdrain, 0)
        plsc.subcore_barrier()

        def _scatter(j, _):
            pltpu.async_copy(val_v.at[j], c_sh.at[flat_v.at[j]], ssem,
                             add=True)
            pltpu.async_copy(val_v.at[j], c_sh.at[gflat_v.at[j]], ssem,
                             add=True)
            return 0
        lax.fori_loop(0, EBUF // NIDX, _scatter, 0)
        pltpu.async_copy(dval_v.at[0], c_sh.at[dflat_v.at[0]], ssem, add=True)
        pltpu.async_copy(dval_v.at[0], c_sh.at[gdflat_v.at[0]], ssem, add=True)

        def _sdrain(j, _):
            pltpu.make_async_copy(val_v.at[j], c_sh.at[flat_v.at[j]],
                                  ssem).wait()
            pltpu.make_async_copy(val_v.at[j], c_sh.at[gflat_v.at[j]],
                                  ssem).wait()
            return 0
        lax.fori_loop(0, EBUF // NIDX, _sdrain, 0)
        pltpu.make_async_copy(dval_v.at[0], c_sh.at[dflat_v.at[0]], ssem).wait()
        pltpu.make_async_copy(dval_v.at[0], c_sh.at[gdflat_v.at[0]], ssem).wait()
        plsc.subcore_barrier()

        def _row_start(r, _):
            row = sid * STRIPE_ROWS + r
            pltpu.async_copy(c_sh.at[pl.ds(row * NP, NP)],
                             out_hbm.at[i - sg_base, row], out_sem)
            return 0
        lax.fori_loop(0, STRIPE_ROWS, _row_start, 0)

        def _row_drain(r, _):
            row = sid * STRIPE_ROWS + r
            pltpu.make_async_copy(c_sh.at[pl.ds(row * NP, NP)],
                                  out_hbm.at[i - sg_base, row], out_sem).wait()
            return 0
        lax.fori_loop(0, STRIPE_ROWS, _row_drain, 0)
        return 0

    lax.fori_loop(0, SG_PER_CORE, _one_subgraph, 0)


def _make_build_counts(sg_base):
    return functools.partial(
        pl.kernel,
        out_type=jax.ShapeDtypeStruct((SG_SPLIT, NP, NP), jnp.float32),
        mesh=plsc.VectorSubcoreMesh(core_axis_name="c", subcore_axis_name="s"),
        scratch_types=[
            pltpu.VMEM((2, EWIN), jnp.int32),
            pltpu.VMEM((EBUF // NIDX, NIDX), jnp.int32),
            pltpu.VMEM((EBUF // NIDX, NIDX), jnp.float32),
            pltpu.VMEM((EBUF // NIDX, NIDX), jnp.int32),
            pltpu.VMEM((1, NIDX), jnp.int32),
            pltpu.VMEM((1, NIDX), jnp.int32),
            pltpu.VMEM((1, NIDX), jnp.float32),
            pltpu.VMEM((ZCHUNK,), jnp.float32),
            pltpu.VMEM_SHARED((CFLAT,), jnp.float32),
            pltpu.SemaphoreType.DMA,
            pltpu.SemaphoreType.DMA,
            pltpu.SemaphoreType.DMA,
            pltpu.SemaphoreType.DMA,
        ],
        name=f"counts_sg{sg_base}",
    )(functools.partial(_sc_counts_body, sg_base))


_build_counts_calls = [
    _make_build_counts(b) for b in range(0, N_SUB, SG_SPLIT)]


def _tc_body(sg_base, x_ref, cen_ref, c_ref, w1_ref, b1_ref, w2_ref, b2_ref,
             w3_ref, b3_ref, wmlo_ref, wmhi_ref, bm_ref, outc_ref, outr_ref,
             xi_ref, hc_ref, hr_ref, dinvw_ref, accc_ref, accr_ref, xsem):
    j = pl.program_id(0)
    r = pl.program_id(1)
    i = j + sg_base

    start = pl.multiple_of(i * N_NODES // 8 * 8, 8)
    xcopy = pltpu.make_async_copy(x_ref.at[pl.ds(start, XR), :], xi_ref, xsem)
    xcopy.start()
    ce = jnp.dot(cen_ref[0], w1_ref[...],
                 preferred_element_type=jnp.float32) + b1_ref[...]
    cr = jnp.dot(ce, wmhi_ref[...],
                 preferred_element_type=jnp.float32) + bm_ref[...]
    xcopy.wait()

    @pl.when(r == 0)
    def _subgraph_head():
        Xi = xi_ref[...]
        M = jnp.maximum(
            jnp.dot(Xi, wmlo_ref[...], preferred_element_type=jnp.float32)
            + cr, 0.0)
        xc = M * Xi
        xr = Xi - xc
        pad = jnp.zeros((NP - XR, D), jnp.bfloat16)
        hc_ref[...] = jnp.concatenate(
            [jnp.dot(xc, w2_ref[...],
                     preferred_element_type=jnp.float32).astype(jnp.bfloat16),
             pad])
        hr_ref[...] = jnp.concatenate(
            [jnp.dot(xr, w3_ref[...],
                     preferred_element_type=jnp.float32).astype(jnp.bfloat16),
             pad])
        degw = c_ref[0, DEGROW - (NRB - 1) * RB:DEGROW - (NRB - 1) * RB + 1, :]
        dinvw_ref[...] = lax.rsqrt(jnp.maximum(degw, 1.0))

    Cb = c_ref[0].astype(jnp.bfloat16)
    deg = jnp.dot(Cb, jnp.ones((NP, 1), jnp.bfloat16),
                  preferred_element_type=jnp.float32)
    dinv = lax.rsqrt(jnp.maximum(deg, 1.0))
    Csc = Cb * dinvw_ref[...].astype(jnp.bfloat16)
    scale = 1.0 / N_SUB
    gc = (dinv * jnp.dot(Csc, hc_ref[...],
                         preferred_element_type=jnp.float32)
          + b2_ref[...]) * scale
    gr = (dinv * jnp.dot(Csc, hr_ref[...],
                         preferred_element_type=jnp.float32)
          + b3_ref[...]) * scale

    off = pl.multiple_of(_rowblk(r) * RB, RB)

    @pl.when(j == 0)
    def _init():
        accc_ref[pl.ds(off, RB), :] = gc
        accr_ref[pl.ds(off, RB), :] = gr

    @pl.when(j > 0)
    def _acc():
        accc_ref[pl.ds(off, RB), :] += gc
        accr_ref[pl.ds(off, RB), :] += gr

    @pl.when((j == SG_SPLIT - 1) & (r == NRB - 1))
    def _flush():
        outc_ref[...] = accc_ref[...]
        outr_ref[...] = accr_ref[...]


def _rowblk(r):
    return NRB - 1 - r


def _tc_half(sg_base, x, centers, C, W1, b1, W2, b2, W3, b3, Wm, bm):
    wfull = pl.BlockSpec((D, D), lambda j, r: (0, 0))
    brow = pl.BlockSpec((1, D), lambda j, r: (0, 0))
    return pl.pallas_call(
        functools.partial(_tc_body, sg_base),
        grid=(SG_SPLIT, NRB),
        in_specs=[
            pl.BlockSpec(memory_space=pltpu.MemorySpace.HBM),
            pl.BlockSpec((1, 1, D), lambda j, r, b=sg_base: (j + b, 0, 0)),
            pl.BlockSpec((1, RB, NP), lambda j, r: (j, _rowblk(r), 0)),
            wfull, brow, wfull, brow, wfull, brow, wfull, wfull, brow,
        ],
        out_specs=[
            pl.BlockSpec((NP, D), lambda j, r: (0, 0)),
            pl.BlockSpec((NP, D), lambda j, r: (0, 0)),
        ],
        out_shape=[
            jax.ShapeDtypeStruct((NP, D), jnp.float32),
            jax.ShapeDtypeStruct((NP, D), jnp.float32),
        ],
        scratch_shapes=[
            pltpu.VMEM((XR, D), jnp.float32),
            pltpu.VMEM((NP, D), jnp.bfloat16),
            pltpu.VMEM((NP, D), jnp.bfloat16),
            pltpu.VMEM((1, NP), jnp.float32),
            pltpu.VMEM((NP, D), jnp.float32),
            pltpu.VMEM((NP, D), jnp.float32),
            pltpu.SemaphoreType.DMA,
        ],
        compiler_params=pltpu.CompilerParams(
            dimension_semantics=("arbitrary", "arbitrary"),
            vmem_limit_bytes=120 * 1024 * 1024),
        name=f"gcn_sg{sg_base}",
    )(x, centers, C, W1, b1.reshape(1, D), W2, b2.reshape(1, D),
      W3, b3.reshape(1, D), Wm[D:], Wm[:D], bm.reshape(1, D))


def kernel(x, edge_index, batch, center_node_indices, W1, b1, W2, b2, W3, b3,
           Wm, bm):
    del batch
    ei = edge_index.astype(jnp.int32)
    Cs = [build(ei) for build in _build_counts_calls]

    center_rows = (jnp.arange(N_SUB, dtype=jnp.int32) * N_NODES
                   + center_node_indices.astype(jnp.int32))
    centers = x[center_rows][:, None, :]

    oc, orr = None, None
    for k, C in enumerate(Cs):
        a, b = _tc_half(k * SG_SPLIT, x, centers, C,
                        W1, b1, W2, b2, W3, b3, Wm, bm)
        oc = a if oc is None else oc + a
        orr = b if orr is None else orr + b
    return oc[:N_CONV], orr[:N_CONV]


# speedup vs baseline: 1.4548x; 1.4548x over previous
import functools

import jax
import jax.numpy as jnp
from jax import lax
from jax.experimental import pallas as pl
from jax.experimental.pallas import tpu as pltpu
from jax.experimental.pallas import tpu_sc as plsc

N_SUB = 8
N_NODES = 1250
N_CONV = N_NODES - 1
E_SUB = 20000
D = 256
NP = 1280
XR = 1256
CFLAT = NP * NP
NCORES = 2
NTILES = 16
CHUNK = 1248
EBUF = 1280
EWIN = EBUF + 128
NIDX = 128
DIAG_PER_TILE = NP // NTILES
STRIPE = CFLAT // NTILES
STRIPE_ROWS = NP // NTILES
ZCHUNK = 12800
SG_SPLIT = 4
RB = 256
NRB = NP // RB
SG_PER_CORE = SG_SPLIT // NCORES
DEGROW = NP - 1


def _sc_counts_body(sg_base, edges_hbm, out_hbm, ev_v, flat_v, val_v,
                    gflat_v, dflat_v, gdflat_v, dval_v, zero_v, c_sh, out_sem,
                    zsem, esem, ssem):
    cid = lax.axis_index("c")
    sid = lax.axis_index("s")
    iota16 = lax.iota(jnp.int32, 16)

    nvalid = jnp.where(sid == NTILES - 1, E_SUB - (NTILES - 1) * CHUNK, CHUNK)

    def _fill_vals(k, _):
        lanes = k * 16 + iota16
        val_v[k // 8, pl.ds((k % 8) * 16, 16)] = jnp.where(
            lanes < nvalid, 1.0, 0.0).astype(jnp.float32)
        return 0
    lax.fori_loop(0, EBUF // 16, _fill_vals, 0)

    def _fill_dvals(k, _):
        lanes = k * 16 + iota16
        dnode = sid * DIAG_PER_TILE + lanes
        dval_v[0, pl.ds(k * 16, 16)] = jnp.where(
            (lanes < DIAG_PER_TILE) & (dnode < N_CONV), 1.0,
            0.0).astype(jnp.float32)
        return 0
    lax.fori_loop(0, NIDX // 16, _fill_dvals, 0)

    def _fill_zero(k, _):
        zero_v[pl.ds(k * 16, 16)] = jnp.zeros((16,), jnp.float32)
        return 0
    lax.fori_loop(0, ZCHUNK // 16, _fill_zero, 0)

    stripe_lo = sid * STRIPE

    def _one_subgraph(g, _):
        i = sg_base + cid + NCORES * g
        node_lo = i * N_NODES
        rshift = lax.rem(node_lo, 8)
        ebase = i * E_SUB + sid * CHUNK
        ebase_al = jnp.minimum(ebase // 128 * 128, N_SUB * E_SUB - EWIN)
        eoff = ebase - ebase_al

        def _zero(j, _):
            pltpu.async_copy(
                zero_v, c_sh.at[pl.ds(stripe_lo + j * ZCHUNK, ZCHUNK)], zsem)
            return 0
        lax.fori_loop(0, STRIPE // ZCHUNK, _zero, 0)
        pltpu.async_copy(edges_hbm.at[:, pl.ds(ebase_al, EWIN)], ev_v, esem)
        pltpu.make_async_copy(edges_hbm.at[:, pl.ds(ebase_al, EWIN)], ev_v,
                              esem).wait()

        def _flat(k, _):
            sl = ev_v[0, pl.ds(eoff + k * 16, 16)] - node_lo
            dl = ev_v[1, pl.ds(eoff + k * 16, 16)] - node_lo
            flat_v[k // 8, pl.ds((k % 8) * 16, 16)] = dl * NP + sl + rshift
            gflat_v[k // 8, pl.ds((k % 8) * 16, 16)] = \
                DEGROW * NP + dl + rshift
            return 0
        lax.fori_loop(0, EBUF // 16, _flat, 0)

        def _fill_dflat(k, _):
            dnode = sid * DIAG_PER_TILE + k * 16 + iota16
            dflat_v[0, pl.ds(k * 16, 16)] = jnp.where(
                dnode < N_CONV, dnode * (NP + 1) + rshift, 0)
            gdflat_v[0, pl.ds(k * 16, 16)] = jnp.where(
                dnode < N_CONV, DEGROW * NP + dnode + rshift, 0)
            return 0
        lax.fori_loop(0, NIDX // 16, _fill_dflat, 0)

        def _zdrain(j, _):
            pltpu.make_async_copy(
                zero_v, c_sh.at[pl.ds(stripe_lo + j * ZCHUNK, ZCHUNK)],
                zsem).wait()
            return 0
        lax.fori_loop(0, STRIPE // ZCHUNK, _zdrain, 0)
        plsc.subcore_barrier()

        def _scatter(j, _):
            pltpu.async_copy(val_v.at[j], c_sh.at[flat_v.at[j]], ssem,
                             add=True)
            pltpu.async_copy(val_v.at[j], c_sh.at[gflat_v.at[j]], ssem,
                             add=True)
            return 0
        lax.fori_loop(0, EBUF // NIDX, _scatter, 0)
        pltpu.async_copy(dval_v.at[0], c_sh.at[dflat_v.at[0]], ssem, add=True)
        pltpu.async_copy(dval_v.at[0], c_sh.at[gdflat_v.at[0]], ssem, add=True)

        def _sdrain(j, _):
            pltpu.make_async_copy(val_v.at[j], c_sh.at[flat_v.at[j]],
                                  ssem).wait()
            pltpu.make_async_copy(val_v.at[j], c_sh.at[gflat_v.at[j]],
                                  ssem).wait()
            return 0
        lax.fori_loop(0, EBUF // NIDX, _sdrain, 0)
        pltpu.make_async_copy(dval_v.at[0], c_sh.at[dflat_v.at[0]], ssem).wait()
        pltpu.make_async_copy(dval_v.at[0], c_sh.at[gdflat_v.at[0]], ssem).wait()
        plsc.subcore_barrier()

        def _row_start(r, _):
            row = sid * STRIPE_ROWS + r
            pltpu.async_copy(c_sh.at[pl.ds(row * NP, NP)],
                             out_hbm.at[i - sg_base, row], out_sem)
            return 0
        lax.fori_loop(0, STRIPE_ROWS, _row_start, 0)

        def _row_drain(r, _):
            row = sid * STRIPE_ROWS + r
            pltpu.make_async_copy(c_sh.at[pl.ds(row * NP, NP)],
                                  out_hbm.at[i - sg_base, row], out_sem).wait()
            return 0
        lax.fori_loop(0, STRIPE_ROWS, _row_drain, 0)
        return 0

    lax.fori_loop(0, SG_PER_CORE, _one_subgraph, 0)


def _make_build_counts(sg_base):
    return functools.partial(
        pl.kernel,
        out_type=jax.ShapeDtypeStruct((SG_SPLIT, NP, NP), jnp.float32),
        mesh=plsc.VectorSubcoreMesh(core_axis_name="c", subcore_axis_name="s"),
        scratch_types=[
            pltpu.VMEM((2, EWIN), jnp.int32),
            pltpu.VMEM((EBUF // NIDX, NIDX), jnp.int32),
            pltpu.VMEM((EBUF // NIDX, NIDX), jnp.float32),
            pltpu.VMEM((EBUF // NIDX, NIDX), jnp.int32),
            pltpu.VMEM((1, NIDX), jnp.int32),
            pltpu.VMEM((1, NIDX), jnp.int32),
            pltpu.VMEM((1, NIDX), jnp.float32),
            pltpu.VMEM((ZCHUNK,), jnp.float32),
            pltpu.VMEM_SHARED((CFLAT,), jnp.float32),
            pltpu.SemaphoreType.DMA,
            pltpu.SemaphoreType.DMA,
            pltpu.SemaphoreType.DMA,
            pltpu.SemaphoreType.DMA,
        ],
        name=f"counts_sg{sg_base}",
    )(functools.partial(_sc_counts_body, sg_base))


_build_counts_calls = [
    _make_build_counts(b) for b in range(0, N_SUB, SG_SPLIT)]


def _tc_body(sg_base, x_ref, cen_ref, c_ref, w1_ref, b1_ref, w2_ref, b2_ref,
             w3_ref, b3_ref, wmlo_ref, wmhi_ref, bm_ref, outc_ref, outr_ref,
             xi_ref, xsem):
    j = pl.program_id(0)
    i = j + sg_base
    start = pl.multiple_of(i * N_NODES // 8 * 8, 8)
    xcopy = pltpu.make_async_copy(x_ref.at[pl.ds(start, XR), :], xi_ref, xsem)
    xcopy.start()
    C = c_ref[0]

    ce = jnp.dot(cen_ref[0], w1_ref[...],
                 preferred_element_type=jnp.float32) + b1_ref[...]
    cr = jnp.dot(ce, wmhi_ref[...],
                 preferred_element_type=jnp.float32) + bm_ref[...]
    xcopy.wait()
    Xi = xi_ref[...]
    M = jnp.maximum(
        jnp.dot(Xi, wmlo_ref[...], preferred_element_type=jnp.float32) + cr,
        0.0)
    xc = M * Xi
    xr = Xi - xc
    pad = jnp.zeros((NP - XR, D), jnp.float32)
    hc = jnp.concatenate(
        [jnp.dot(xc, w2_ref[...], preferred_element_type=jnp.float32), pad])
    hr = jnp.concatenate(
        [jnp.dot(xr, w3_ref[...], preferred_element_type=jnp.float32), pad])

    Cb = C.astype(jnp.bfloat16)
    deg = jnp.dot(Cb, jnp.ones((NP, 1), jnp.bfloat16),
                  preferred_element_type=jnp.float32)
    dinv = lax.rsqrt(jnp.maximum(deg, 1.0))
    degw = C[DEGROW:DEGROW + 1, :]
    dinvw = lax.rsqrt(jnp.maximum(degw, 1.0))
    Csc = Cb * dinvw.astype(jnp.bfloat16)
    gc = dinv * jnp.dot(Csc, hc.astype(jnp.bfloat16),
                        preferred_element_type=jnp.float32) + b2_ref[...]
    gr = dinv * jnp.dot(Csc, hr.astype(jnp.bfloat16),
                        preferred_element_type=jnp.float32) + b3_ref[...]

    scale = 1.0 / N_SUB
    gc = gc * scale
    gr = gr * scale

    @pl.when(j == 0)
    def _init():
        outc_ref[...] = gc
        outr_ref[...] = gr

    @pl.when(j > 0)
    def _acc():
        outc_ref[...] += gc
        outr_ref[...] += gr


def _tc_half(sg_base, x, centers, C, W1, b1, W2, b2, W3, b3, Wm, bm):
    wfull = pl.BlockSpec((D, D), lambda j: (0, 0))
    brow = pl.BlockSpec((1, D), lambda j: (0, 0))
    return pl.pallas_call(
        functools.partial(_tc_body, sg_base),
        grid=(SG_SPLIT,),
        in_specs=[
            pl.BlockSpec(memory_space=pltpu.MemorySpace.HBM),
            pl.BlockSpec((1, 1, D), lambda j, b=sg_base: (j + b, 0, 0)),
            pl.BlockSpec((1, NP, NP), lambda j: (j, 0, 0)),
            wfull, brow, wfull, brow, wfull, brow, wfull, wfull, brow,
        ],
        out_specs=[
            pl.BlockSpec((NP, D), lambda j: (0, 0)),
            pl.BlockSpec((NP, D), lambda j: (0, 0)),
        ],
        out_shape=[
            jax.ShapeDtypeStruct((NP, D), jnp.float32),
            jax.ShapeDtypeStruct((NP, D), jnp.float32),
        ],
        scratch_shapes=[
            pltpu.VMEM((XR, D), jnp.float32),
            pltpu.SemaphoreType.DMA,
        ],
        compiler_params=pltpu.CompilerParams(
            dimension_semantics=("arbitrary",),
            vmem_limit_bytes=120 * 1024 * 1024),
        name=f"gcn_sg{sg_base}",
    )(x, centers, C, W1, b1.reshape(1, D), W2, b2.reshape(1, D),
      W3, b3.reshape(1, D), Wm[D:], Wm[:D], bm.reshape(1, D))


def kernel(x, edge_index, batch, center_node_indices, W1, b1, W2, b2, W3, b3,
           Wm, bm):
    del batch
    ei = edge_index.astype(jnp.int32)
    Cs = [build(ei) for build in _build_counts_calls]

    center_rows = (jnp.arange(N_SUB, dtype=jnp.int32) * N_NODES
                   + center_node_indices.astype(jnp.int32))
    centers = x[center_rows][:, None, :]

    oc, orr = None, None
    for k, C in enumerate(Cs):
        a, b = _tc_half(k * SG_SPLIT, x, centers, C,
                        W1, b1, W2, b2, W3, b3, Wm, bm)
        oc = a if oc is None else oc + a
        orr = b if orr is None else orr + b
    return oc[:N_CONV], orr[:N_CONV]


# speedup vs baseline: 1.4574x; 1.0018x over previous
import functools

import jax
import jax.numpy as jnp
from jax import lax
from jax.experimental import pallas as pl
from jax.experimental.pallas import tpu as pltpu
from jax.experimental.pallas import tpu_sc as plsc

N_SUB = 8
N_NODES = 1250
N_CONV = N_NODES - 1
E_SUB = 20000
D = 256
NP = 1280
XR = 1256
CFLAT = NP * NP
NCORES = 2
NTILES = 16
CHUNK = 1248
EBUF = 1280
EWIN = EBUF + 128
NIDX = 128
DIAG_PER_TILE = NP // NTILES
STRIPE = CFLAT // NTILES
STRIPE_ROWS = NP // NTILES
ZCHUNK = 12800
SG_SPLIT = 4
SG_PER_CORE = SG_SPLIT // NCORES
DEGROW = NP - 1


def _sc_counts_body(sg_base, edges_hbm, out_hbm, ev_v, flat_v, val_v,
                    gflat_v, dflat_v, gdflat_v, dval_v, zero_v, c_sh, out_sem,
                    zsem, esem, ssem):
    cid = lax.axis_index("c")
    sid = lax.axis_index("s")
    iota16 = lax.iota(jnp.int32, 16)

    nvalid = jnp.where(sid == NTILES - 1, E_SUB - (NTILES - 1) * CHUNK, CHUNK)

    def _fill_vals(k, _):
        lanes = k * 16 + iota16
        val_v[k // 8, pl.ds((k % 8) * 16, 16)] = jnp.where(
            lanes < nvalid, 1.0, 0.0).astype(jnp.float32)
        return 0
    lax.fori_loop(0, EBUF // 16, _fill_vals, 0)

    def _fill_dvals(k, _):
        lanes = k * 16 + iota16
        dnode = sid * DIAG_PER_TILE + lanes
        dval_v[0, pl.ds(k * 16, 16)] = jnp.where(
            (lanes < DIAG_PER_TILE) & (dnode < N_CONV), 1.0,
            0.0).astype(jnp.float32)
        return 0
    lax.fori_loop(0, NIDX // 16, _fill_dvals, 0)

    def _fill_zero(k, _):
        zero_v[pl.ds(k * 16, 16)] = jnp.zeros((16,), jnp.float32)
        return 0
    lax.fori_loop(0, ZCHUNK // 16, _fill_zero, 0)

    stripe_lo = sid * STRIPE

    def _one_subgraph(g, _):
        i = sg_base + cid + NCORES * g
        node_lo = i * N_NODES
        rshift = lax.rem(node_lo, 8)
        ebase = i * E_SUB + sid * CHUNK
        ebase_al = jnp.minimum(ebase // 128 * 128, N_SUB * E_SUB - EWIN)
        eoff = ebase - ebase_al

        def _zero(j, _):
            pltpu.async_copy(
                zero_v, c_sh.at[pl.ds(stripe_lo + j * ZCHUNK, ZCHUNK)], zsem)
            return 0
        lax.fori_loop(0, STRIPE // ZCHUNK, _zero, 0)
        pltpu.async_copy(edges_hbm.at[:, pl.ds(ebase_al, EWIN)], ev_v, esem)
        pltpu.make_async_copy(edges_hbm.at[:, pl.ds(ebase_al, EWIN)], ev_v,
                              esem).wait()

        def _flat(k, _):
            sl = ev_v[0, pl.ds(eoff + k * 16, 16)] - node_lo
            dl = ev_v[1, pl.ds(eoff + k * 16, 16)] - node_lo
            flat_v[k // 8, pl.ds((k % 8) * 16, 16)] = dl * NP + sl + rshift
            gflat_v[k // 8, pl.ds((k % 8) * 16, 16)] = \
                DEGROW * NP + dl + rshift
            return 0
        lax.fori_loop(0, EBUF // 16, _flat, 0)

        def _fill_dflat(k, _):
            dnode = sid * DIAG_PER_TILE + k * 16 + iota16
            dflat_v[0, pl.ds(k * 16, 16)] = jnp.where(
                dnode < N_CONV, dnode * (NP + 1) + rshift, 0)
            gdflat_v[0, pl.ds(k * 16, 16)] = jnp.where(
                dnode < N_CONV, DEGROW * NP + dnode + rshift, 0)
            return 0
        lax.fori_loop(0, NIDX // 16, _fill_dflat, 0)

        def _zdrain(j, _):
            pltpu.make_async_copy(
                zero_v, c_sh.at[pl.ds(stripe_lo + j * ZCHUNK, ZCHUNK)],
                zsem).wait()
            return 0
        lax.fori_loop(0, STRIPE // ZCHUNK, _zdrain, 0)
        plsc.subcore_barrier()

        def _scatter(j, _):
            pltpu.async_copy(val_v.at[j], c_sh.at[flat_v.at[j]], ssem,
                             add=True)
            pltpu.async_copy(val_v.at[j], c_sh.at[gflat_v.at[j]], ssem,
                             add=True)
            return 0
        lax.fori_loop(0, EBUF // NIDX, _scatter, 0)
        pltpu.async_copy(dval_v.at[0], c_sh.at[dflat_v.at[0]], ssem, add=True)
        pltpu.async_copy(dval_v.at[0], c_sh.at[gdflat_v.at[0]], ssem, add=True)

        def _sdrain(j, _):
            pltpu.make_async_copy(val_v.at[j], c_sh.at[flat_v.at[j]],
                                  ssem).wait()
            pltpu.make_async_copy(val_v.at[j], c_sh.at[gflat_v.at[j]],
                                  ssem).wait()
            return 0
        lax.fori_loop(0, EBUF // NIDX, _sdrain, 0)
        pltpu.make_async_copy(dval_v.at[0], c_sh.at[dflat_v.at[0]], ssem).wait()
        pltpu.make_async_copy(dval_v.at[0], c_sh.at[gdflat_v.at[0]], ssem).wait()
        plsc.subcore_barrier()

        def _row_start(r, _):
            row = sid * STRIPE_ROWS + r
            pltpu.async_copy(c_sh.at[pl.ds(row * NP, NP)],
                             out_hbm.at[i - sg_base, row], out_sem)
            return 0
        lax.fori_loop(0, STRIPE_ROWS, _row_start, 0)

        def _row_drain(r, _):
            row = sid * STRIPE_ROWS + r
            pltpu.make_async_copy(c_sh.at[pl.ds(row * NP, NP)],
                                  out_hbm.at[i - sg_base, row], out_sem).wait()
            return 0
        lax.fori_loop(0, STRIPE_ROWS, _row_drain, 0)
        return 0

    lax.fori_loop(0, SG_PER_CORE, _one_subgraph, 0)


def _make_build_counts(sg_base):
    return functools.partial(
        pl.kernel,
        out_type=jax.ShapeDtypeStruct((SG_SPLIT, NP, NP), jnp.float32),
        mesh=plsc.VectorSubcoreMesh(core_axis_name="c", subcore_axis_name="s",
                                    num_cores=NCORES, num_subcores=NTILES),
        scratch_types=[
            pltpu.VMEM((2, EWIN), jnp.int32),
            pltpu.VMEM((EBUF // NIDX, NIDX), jnp.int32),
            pltpu.VMEM((EBUF // NIDX, NIDX), jnp.float32),
            pltpu.VMEM((EBUF // NIDX, NIDX), jnp.int32),
            pltpu.VMEM((1, NIDX), jnp.int32),
            pltpu.VMEM((1, NIDX), jnp.int32),
            pltpu.VMEM((1, NIDX), jnp.float32),
            pltpu.VMEM((ZCHUNK,), jnp.float32),
            pltpu.VMEM_SHARED((CFLAT,), jnp.float32),
            pltpu.SemaphoreType.DMA,
            pltpu.SemaphoreType.DMA,
            pltpu.SemaphoreType.DMA,
            pltpu.SemaphoreType.DMA,
        ],
        name=f"counts_sg{sg_base}",
    )(functools.partial(_sc_counts_body, sg_base))


_build_counts_calls = [
    _make_build_counts(b) for b in range(0, N_SUB, SG_SPLIT)]


def _tc_body(sg_base, x_ref, cen_ref, c_ref, w1_ref, b1_ref, w2_ref, b2_ref,
             w3_ref, b3_ref, wmlo_ref, wmhi_ref, bm_ref, outc_ref, outr_ref,
             xi_ref, xsem):
    j = pl.program_id(0)
    i = j + sg_base
    start = pl.multiple_of(i * N_NODES // 8 * 8, 8)
    xcopy = pltpu.make_async_copy(x_ref.at[pl.ds(start, XR), :], xi_ref, xsem)
    xcopy.start()
    C = c_ref[0]

    ce = jnp.dot(cen_ref[0], w1_ref[...],
                 preferred_element_type=jnp.float32) + b1_ref[...]
    cr = jnp.dot(ce, wmhi_ref[...],
                 preferred_element_type=jnp.float32) + bm_ref[...]
    xcopy.wait()
    Xi = xi_ref[...]
    M = jnp.maximum(
        jnp.dot(Xi, wmlo_ref[...], preferred_element_type=jnp.float32) + cr,
        0.0)
    xc = M * Xi
    xr = Xi - xc
    pad = jnp.zeros((NP - XR, D), jnp.float32)
    hc = jnp.concatenate(
        [jnp.dot(xc, w2_ref[...], preferred_element_type=jnp.float32), pad])
    hr = jnp.concatenate(
        [jnp.dot(xr, w3_ref[...], preferred_element_type=jnp.float32), pad])

    Cb = C.astype(jnp.bfloat16)
    deg = jnp.dot(Cb, jnp.ones((NP, 1), jnp.bfloat16),
                  preferred_element_type=jnp.float32)
    dinv = lax.rsqrt(jnp.maximum(deg, 1.0))
    degw = C[DEGROW:DEGROW + 1, :]
    dinvw = lax.rsqrt(jnp.maximum(degw, 1.0))
    Csc = Cb * dinvw.astype(jnp.bfloat16)
    gc = dinv * jnp.dot(Csc, hc.astype(jnp.bfloat16),
                        preferred_element_type=jnp.float32) + b2_ref[...]
    gr = dinv * jnp.dot(Csc, hr.astype(jnp.bfloat16),
                        preferred_element_type=jnp.float32) + b3_ref[...]

    scale = 1.0 / N_SUB
    gc = gc * scale
    gr = gr * scale

    @pl.when(j == 0)
    def _init():
        outc_ref[...] = gc
        outr_ref[...] = gr

    @pl.when(j > 0)
    def _acc():
        outc_ref[...] += gc
        outr_ref[...] += gr


def _tc_half(sg_base, x, centers, C, W1, b1, W2, b2, W3, b3, Wm, bm):
    wfull = pl.BlockSpec((D, D), lambda j: (0, 0))
    brow = pl.BlockSpec((1, D), lambda j: (0, 0))
    return pl.pallas_call(
        functools.partial(_tc_body, sg_base),
        grid=(SG_SPLIT,),
        in_specs=[
            pl.BlockSpec(memory_space=pltpu.MemorySpace.HBM),
            pl.BlockSpec((1, 1, D), lambda j, b=sg_base: (j + b, 0, 0)),
            pl.BlockSpec((1, NP, NP), lambda j: (j, 0, 0)),
            wfull, brow, wfull, brow, wfull, brow, wfull, wfull, brow,
        ],
        out_specs=[
            pl.BlockSpec((NP, D), lambda j: (0, 0)),
            pl.BlockSpec((NP, D), lambda j: (0, 0)),
        ],
        out_shape=[
            jax.ShapeDtypeStruct((NP, D), jnp.float32),
            jax.ShapeDtypeStruct((NP, D), jnp.float32),
        ],
        scratch_shapes=[
            pltpu.VMEM((XR, D), jnp.float32),
            pltpu.SemaphoreType.DMA,
        ],
        compiler_params=pltpu.CompilerParams(
            dimension_semantics=("arbitrary",),
            vmem_limit_bytes=120 * 1024 * 1024),
        name=f"gcn_sg{sg_base}",
    )(x, centers, C, W1, b1.reshape(1, D), W2, b2.reshape(1, D),
      W3, b3.reshape(1, D), Wm[D:], Wm[:D], bm.reshape(1, D))


def kernel(x, edge_index, batch, center_node_indices, W1, b1, W2, b2, W3, b3,
           Wm, bm):
    del batch
    ei = edge_index.astype(jnp.int32)
    Cs = [build(ei) for build in _build_counts_calls]

    center_rows = (jnp.arange(N_SUB, dtype=jnp.int32) * N_NODES
                   + center_node_indices.astype(jnp.int32))
    centers = x[center_rows][:, None, :]

    oc, orr = None, None
    for k, C in enumerate(Cs):
        a, b = _tc_half(k * SG_SPLIT, x, centers, C,
                        W1, b1, W2, b2, W3, b3, Wm, bm)
        oc = a if oc is None else oc + a
        orr = b if orr is None else orr + b
    return oc[:N_CONV], orr[:N_CONV]
